# baseline (speedup 1.0000x reference)
import dataclasses

import jax
import jax.numpy as jnp
from jax import lax
from jax.experimental import pallas as pl
from jax.experimental.pallas import tpu as pltpu
from jax.experimental.pallas import tpu_sc as plsc

_EPS = 1e-6
_NBINS = 100
_D = 128
_BP = 128
_R = 8192

_SC_WORKERS = 32


def _bucketize_body(y_hbm, edges_hbm, idx_hbm, y_v, e_v, idx_v):
    rows = y_v.shape[0]
    wid = lax.axis_index("c") * 16 + lax.axis_index("s")
    base = wid * rows
    pltpu.sync_copy(edges_hbm, e_v)
    pltpu.sync_copy(y_hbm.at[pl.ds(base, rows)], y_v)

    @pl.loop(0, rows, step=128)
    def _(i):
        for u in range(0, 128, 16):
            y = y_v[pl.ds(i + u, 16)]
            a = (y * float(_NBINS)).astype(jnp.int32)
            a = jnp.minimum(jnp.maximum(a, 0), _NBINS - 1)
            e_a = plsc.load_gather(e_v, [a])
            e_a1 = plsc.load_gather(e_v, [a + 1])
            idx = a - (y < e_a).astype(jnp.int32) + (y >= e_a1).astype(jnp.int32)
            idx = jnp.minimum(jnp.maximum(idx, 0), _NBINS - 1)
            idx_v[pl.ds(i + u, 16)] = idx

    pltpu.sync_copy(idx_v, idx_hbm.at[pl.ds(base, rows)])


def _prep_body(rm_ref, rv_ref, sm_ref, sv_ref, hi_ref):
    scale = jnp.sqrt(sv_ref[...] + _EPS) / jnp.sqrt(rv_ref[...] + _EPS)
    offset = sm_ref[...] - rm_ref[...] * scale
    comb = jnp.concatenate([scale, offset], axis=1)
    hi = comb.astype(jnp.bfloat16)
    lo = (comb - hi.astype(jnp.float32)).astype(jnp.bfloat16)
    hi_ref[...] = jnp.concatenate([hi, lo], axis=0)


_CH = 2048


def _main_body(idx_ref, hilo_ref, z_ref, out_ref):
    lanes = lax.broadcasted_iota(jnp.int32, (1, _BP), 1)
    hilo = hilo_ref[...]
    for c in range(_R // _CH):
        idx16 = idx_ref[pl.ds(c * (_CH // _D), _CH // _D), :]
        slabs = []
        for g in range(_CH // _D):
            ic = jnp.transpose(idx16[g : g + 1, :])
            slabs.append((ic == lanes).astype(jnp.bfloat16))
        oh = jnp.concatenate(slabs, axis=0)
        oh2 = jnp.concatenate([oh, oh], axis=1)
        so = jnp.dot(oh2, hilo, preferred_element_type=jnp.float32)
        rows = pl.ds(c * _CH, _CH)
        out_ref[rows, :] = z_ref[rows, :] * so[:, :_D] + so[:, _D:]


def kernel(z, y_gene, edges, running_mean, running_var, smoothed_mean, smoothed_var):
    n, d = z.shape

    sc_params = pltpu.CompilerParams()
    if "needs_layout_passes" in pltpu.CompilerParams.__dataclass_fields__:
        sc_params = dataclasses.replace(sc_params, needs_layout_passes=False)

    edges_padded = jnp.pad(edges, (0, 128 - (_NBINS + 1)), constant_values=jnp.inf)
    idx = pl.kernel(
        _bucketize_body,
        out_type=jax.ShapeDtypeStruct((n,), jnp.int32),
        mesh=plsc.VectorSubcoreMesh(core_axis_name="c", subcore_axis_name="s"),
        scratch_types=[
            pltpu.VMEM((n // _SC_WORKERS,), jnp.float32),
            pltpu.VMEM((128,), jnp.float32),
            pltpu.VMEM((n // _SC_WORKERS,), jnp.int32),
        ],
        compiler_params=sc_params,
    )(y_gene, edges_padded)

    pad = ((0, _BP - _NBINS), (0, 0))
    rm = jnp.pad(running_mean, pad)
    rv = jnp.pad(running_var, pad, constant_values=1.0)
    sm = jnp.pad(smoothed_mean, pad)
    sv = jnp.pad(smoothed_var, pad, constant_values=1.0)

    hilo = pl.pallas_call(
        _prep_body,
        out_shape=jax.ShapeDtypeStruct((2 * _BP, 2 * _D), jnp.bfloat16),
    )(rm, rv, sm, sv)

    idx2d = idx.reshape(n // _D, _D)
    out = pl.pallas_call(
        _main_body,
        grid=(n // _R,),
        in_specs=[
            pl.BlockSpec((_R // _D, _D), lambda i: (i, 0)),
            pl.BlockSpec((2 * _BP, 2 * _D), lambda i: (0, 0)),
            pl.BlockSpec((_R, _D), lambda i: (i, 0)),
        ],
        out_specs=pl.BlockSpec((_R, _D), lambda i: (i, 0)),
        out_shape=jax.ShapeDtypeStruct((n, d), jnp.float32),
    )(idx2d, hilo, z)
    return out


# speedup vs baseline: 1.0715x; 1.0715x over previous
import dataclasses

import jax
import jax.numpy as jnp
from jax import lax
from jax.experimental import pallas as pl
from jax.experimental.pallas import tpu as pltpu
from jax.experimental.pallas import tpu_sc as plsc

_EPS = 1e-6
_NBINS = 100
_D = 128
_BP = 128
_R = 16384

_SC_WORKERS = 32


def _bucketize_body(y_hbm, edges_hbm, idx_hbm, y_v, e_v, idx_v):
    rows = y_v.shape[0]
    wid = lax.axis_index("c") * 16 + lax.axis_index("s")
    base = y_hbm.shape[0] // 2 + wid * rows
    pltpu.sync_copy(edges_hbm, e_v)
    pltpu.sync_copy(y_hbm.at[pl.ds(base, rows)], y_v)

    @pl.loop(0, rows, step=128)
    def _(i):
        for u in range(0, 128, 16):
            y = y_v[pl.ds(i + u, 16)]
            a = (y * float(_NBINS)).astype(jnp.int32)
            a = jnp.minimum(jnp.maximum(a, 0), _NBINS - 1)
            e_a = plsc.load_gather(e_v, [a])
            e_a1 = plsc.load_gather(e_v, [a + 1])
            idx = a - (y < e_a).astype(jnp.int32) + (y >= e_a1).astype(jnp.int32)
            idx = jnp.minimum(jnp.maximum(idx, 0), _NBINS - 1)
            idx_v[pl.ds(i + u, 16)] = idx

    pltpu.sync_copy(idx_v, idx_hbm.at[pl.ds(wid * rows, rows)])


def _prep_body(rm_ref, rv_ref, sm_ref, sv_ref, hi_ref):
    scale = jnp.sqrt(sv_ref[...] + _EPS) / jnp.sqrt(rv_ref[...] + _EPS)
    offset = sm_ref[...] - rm_ref[...] * scale
    comb = jnp.concatenate([scale, offset], axis=1)
    hi = comb.astype(jnp.bfloat16)
    lo = (comb - hi.astype(jnp.float32)).astype(jnp.bfloat16)
    hi_ref[...] = jnp.concatenate([hi, lo], axis=0)


_CH = 2048


def _main_body_y(y_ref, edges_ref, hilo_ref, z_ref, out_ref):
    e = edges_ref[...]
    hilo = hilo_ref[...]
    for c in range(_R // _CH):
        y16 = y_ref[pl.ds(c * (_CH // _D), _CH // _D), :]
        slabs = []
        for g in range(_CH // _D):
            yc = jnp.transpose(y16[g : g + 1, :])
            slabs.append(
                jnp.logical_and(yc >= e[0:1, :], yc < e[1:2, :]).astype(jnp.bfloat16)
            )
        oh = jnp.concatenate(slabs, axis=0)
        oh2 = jnp.concatenate([oh, oh], axis=1)
        so = jnp.dot(oh2, hilo, preferred_element_type=jnp.float32)
        rows = pl.ds(c * _CH, _CH)
        out_ref[rows, :] = z_ref[rows, :] * so[:, :_D] + so[:, _D:]


def _main_body_idx(idx_ref, hilo_ref, z_ref, prev_ref, out_ref):
    lanes = lax.broadcasted_iota(jnp.int32, (1, _BP), 1)
    hilo = hilo_ref[...]
    for c in range(_R // _CH):
        idx16 = idx_ref[pl.ds(c * (_CH // _D), _CH // _D), :]
        slabs = []
        for g in range(_CH // _D):
            ic = jnp.transpose(idx16[g : g + 1, :])
            slabs.append((ic == lanes).astype(jnp.bfloat16))
        oh = jnp.concatenate(slabs, axis=0)
        oh2 = jnp.concatenate([oh, oh], axis=1)
        so = jnp.dot(oh2, hilo, preferred_element_type=jnp.float32)
        rows = pl.ds(c * _CH, _CH)
        out_ref[rows, :] = z_ref[rows, :] * so[:, :_D] + so[:, _D:]


def kernel(z, y_gene, edges, running_mean, running_var, smoothed_mean, smoothed_var):
    n, d = z.shape

    sc_params = pltpu.CompilerParams()
    if "needs_layout_passes" in pltpu.CompilerParams.__dataclass_fields__:
        sc_params = dataclasses.replace(sc_params, needs_layout_passes=False)

    edges_padded = jnp.pad(edges, (0, 128 - (_NBINS + 1)), constant_values=jnp.inf)
    sc_rows = (n // 2) // _SC_WORKERS
    idx = pl.kernel(
        _bucketize_body,
        out_type=jax.ShapeDtypeStruct((n // 2,), jnp.int32),
        mesh=plsc.VectorSubcoreMesh(core_axis_name="c", subcore_axis_name="s"),
        scratch_types=[
            pltpu.VMEM((sc_rows,), jnp.float32),
            pltpu.VMEM((128,), jnp.float32),
            pltpu.VMEM((sc_rows,), jnp.int32),
        ],
        compiler_params=sc_params,
    )(y_gene, edges_padded)

    pad = ((0, _BP - _NBINS), (0, 0))
    rm = jnp.pad(running_mean, pad)
    rv = jnp.pad(running_var, pad, constant_values=1.0)
    sm = jnp.pad(smoothed_mean, pad)
    sv = jnp.pad(smoothed_var, pad, constant_values=1.0)

    hilo = pl.pallas_call(
        _prep_body,
        out_shape=jax.ShapeDtypeStruct((2 * _BP, 2 * _D), jnp.bfloat16),
    )(rm, rv, sm, sv)

    edges_pad = jnp.full((2, _BP), jnp.inf, dtype=jnp.float32)
    edges_pad = edges_pad.at[0, :_NBINS].set(edges[:_NBINS])
    edges_pad = edges_pad.at[1, :_NBINS].set(edges[1 : _NBINS + 1])

    half = n // 2
    g_half = half // _R
    y2d = y_gene.reshape(n // _D, _D)

    out1 = pl.pallas_call(
        _main_body_y,
        grid=(g_half,),
        in_specs=[
            pl.BlockSpec((_R // _D, _D), lambda i: (i, 0)),
            pl.BlockSpec((2, _BP), lambda i: (0, 0)),
            pl.BlockSpec((2 * _BP, 2 * _D), lambda i: (0, 0)),
            pl.BlockSpec((_R, _D), lambda i: (i, 0)),
        ],
        out_specs=pl.BlockSpec((_R, _D), lambda i: (i, 0)),
        out_shape=jax.ShapeDtypeStruct((n, d), jnp.float32),
    )(y2d, edges_pad, hilo, z)

    idx2d = idx.reshape(half // _D, _D)
    out = pl.pallas_call(
        _main_body_idx,
        grid=(g_half,),
        in_specs=[
            pl.BlockSpec((_R // _D, _D), lambda i: (i, 0)),
            pl.BlockSpec((2 * _BP, 2 * _D), lambda i: (0, 0)),
            pl.BlockSpec((_R, _D), lambda i: (i + g_half, 0)),
            pl.BlockSpec(memory_space=pl.ANY),
        ],
        out_specs=pl.BlockSpec((_R, _D), lambda i: (i + g_half, 0)),
        out_shape=jax.ShapeDtypeStruct((n, d), jnp.float32),
        input_output_aliases={3: 0},
    )(idx2d, hilo, z, out1)
    return out


# speedup vs baseline: 1.0889x; 1.0162x over previous
import dataclasses

import jax
import jax.numpy as jnp
from jax import lax
from jax.experimental import pallas as pl
from jax.experimental.pallas import tpu as pltpu
from jax.experimental.pallas import tpu_sc as plsc

_EPS = 1e-6
_NBINS = 100
_D = 128
_BP = 128
_R = 16384

_SC_WORKERS = 32


def _bucketize_body(y_hbm, edges_hbm, idx_hbm, y_v, e_v, idx_v):
    rows = y_v.shape[0]
    wid = lax.axis_index("c") * 16 + lax.axis_index("s")
    base = wid * rows
    pltpu.sync_copy(edges_hbm, e_v)
    pltpu.sync_copy(y_hbm.at[pl.ds(base, rows)], y_v)

    @pl.loop(0, rows, step=128)
    def _(i):
        for u in range(0, 128, 16):
            y = y_v[pl.ds(i + u, 16)]
            a = (y * float(_NBINS)).astype(jnp.int32)
            a = jnp.minimum(jnp.maximum(a, 0), _NBINS - 1)
            e_a = plsc.load_gather(e_v, [a])
            e_a1 = plsc.load_gather(e_v, [a + 1])
            idx = a - (y < e_a).astype(jnp.int32) + (y >= e_a1).astype(jnp.int32)
            idx = jnp.minimum(jnp.maximum(idx, 0), _NBINS - 1)
            idx_v[pl.ds(i + u, 16)] = idx

    pltpu.sync_copy(idx_v, idx_hbm.at[pl.ds(base, rows)])


def _prep_body(rm_ref, rv_ref, sm_ref, sv_ref, hi_ref):
    scale = jnp.sqrt(sv_ref[...] + _EPS) / jnp.sqrt(rv_ref[...] + _EPS)
    offset = sm_ref[...] - rm_ref[...] * scale
    comb = jnp.concatenate([scale, offset], axis=1)
    hi = comb.astype(jnp.bfloat16)
    lo = (comb - hi.astype(jnp.float32)).astype(jnp.bfloat16)
    hi_ref[...] = jnp.concatenate([hi, lo], axis=0)


_CH = 2048


def _main_body(idx_ref, hilo_ref, z_ref, out_ref):
    lanes = lax.broadcasted_iota(jnp.int32, (1, _BP), 1)
    hilo = hilo_ref[...]
    for c in range(_R // _CH):
        idx16 = idx_ref[pl.ds(c * (_CH // _D), _CH // _D), :]
        slabs = []
        for g in range(_CH // _D):
            ic = jnp.transpose(idx16[g : g + 1, :])
            slabs.append((ic == lanes).astype(jnp.bfloat16))
        oh = jnp.concatenate(slabs, axis=0)
        oh2 = jnp.concatenate([oh, oh], axis=1)
        so = jnp.dot(oh2, hilo, preferred_element_type=jnp.float32)
        rows = pl.ds(c * _CH, _CH)
        out_ref[rows, :] = z_ref[rows, :] * so[:, :_D] + so[:, _D:]


def kernel(z, y_gene, edges, running_mean, running_var, smoothed_mean, smoothed_var):
    n, d = z.shape

    sc_params = pltpu.CompilerParams()
    if "needs_layout_passes" in pltpu.CompilerParams.__dataclass_fields__:
        sc_params = dataclasses.replace(sc_params, needs_layout_passes=False)

    edges_padded = jnp.pad(edges, (0, 128 - (_NBINS + 1)), constant_values=jnp.inf)
    idx = pl.kernel(
        _bucketize_body,
        out_type=jax.ShapeDtypeStruct((n,), jnp.int32),
        mesh=plsc.VectorSubcoreMesh(core_axis_name="c", subcore_axis_name="s"),
        scratch_types=[
            pltpu.VMEM((n // _SC_WORKERS,), jnp.float32),
            pltpu.VMEM((128,), jnp.float32),
            pltpu.VMEM((n // _SC_WORKERS,), jnp.int32),
        ],
        compiler_params=sc_params,
    )(y_gene, edges_padded)

    pad = ((0, _BP - _NBINS), (0, 0))
    rm = jnp.pad(running_mean, pad)
    rv = jnp.pad(running_var, pad, constant_values=1.0)
    sm = jnp.pad(smoothed_mean, pad)
    sv = jnp.pad(smoothed_var, pad, constant_values=1.0)

    hilo = pl.pallas_call(
        _prep_body,
        out_shape=jax.ShapeDtypeStruct((2 * _BP, 2 * _D), jnp.bfloat16),
    )(rm, rv, sm, sv)

    idx2d = idx.reshape(n // _D, _D)
    out = pl.pallas_call(
        _main_body,
        grid=(n // _R,),
        in_specs=[
            pl.BlockSpec((_R // _D, _D), lambda i: (i, 0)),
            pl.BlockSpec((2 * _BP, 2 * _D), lambda i: (0, 0)),
            pl.BlockSpec((_R, _D), lambda i: (i, 0)),
        ],
        out_specs=pl.BlockSpec((_R, _D), lambda i: (i, 0)),
        out_shape=jax.ShapeDtypeStruct((n, d), jnp.float32),
    )(idx2d, hilo, z)
    return out


# speedup vs baseline: 1.1038x; 1.0137x over previous
import dataclasses

import jax
import jax.numpy as jnp
from jax import lax
from jax.experimental import pallas as pl
from jax.experimental.pallas import tpu as pltpu
from jax.experimental.pallas import tpu_sc as plsc

_EPS = 1e-6
_NBINS = 100
_D = 128
_BP = 128
_R = 16384

_SC_WORKERS = 32


def _bucketize_body(y_hbm, edges_hbm, idx_hbm, y_v, e_v, idx_v):
    rows = y_v.shape[0]
    wid = lax.axis_index("c") * 16 + lax.axis_index("s")
    base = wid * rows
    pltpu.sync_copy(edges_hbm, e_v)
    pltpu.sync_copy(y_hbm.at[pl.ds(base, rows)], y_v)

    @pl.loop(0, rows, step=128)
    def _(i):
        for u in range(0, 128, 16):
            y = y_v[pl.ds(i + u, 16)]
            a = (y * float(_NBINS)).astype(jnp.int32)
            a = jnp.minimum(jnp.maximum(a, 0), _NBINS - 1)
            e_a = plsc.load_gather(e_v, [a])
            e_a1 = plsc.load_gather(e_v, [a + 1])
            idx = a - (y < e_a).astype(jnp.int32) + (y >= e_a1).astype(jnp.int32)
            idx = jnp.minimum(jnp.maximum(idx, 0), _NBINS - 1)
            idx_v[pl.ds(i + u, 16)] = idx

    pltpu.sync_copy(idx_v, idx_hbm.at[pl.ds(base, rows)])


def _prep_body(rm_ref, rv_ref, sm_ref, sv_ref, hi_ref):
    scale = jnp.sqrt(sv_ref[...] + _EPS) / jnp.sqrt(rv_ref[...] + _EPS)
    offset = sm_ref[...] - rm_ref[...] * scale
    comb = jnp.concatenate([scale, offset], axis=1)
    hi = comb.astype(jnp.bfloat16)
    lo = (comb - hi.astype(jnp.float32)).astype(jnp.bfloat16)
    hi_ref[...] = jnp.concatenate([hi, lo], axis=0)


_CH = 2048


def _main_body(idx_ref, rm_ref, rv_ref, sm_ref, sv_ref, z_ref, out_ref, hilo_ref):
    @pl.when(pl.program_id(0) == 0)
    def _():
        _prep_body(rm_ref, rv_ref, sm_ref, sv_ref, hilo_ref)

    lanes = lax.broadcasted_iota(jnp.int32, (1, _BP), 1)
    hilo = hilo_ref[...]
    for c in range(_R // _CH):
        idx16 = idx_ref[pl.ds(c * (_CH // _D), _CH // _D), :]
        slabs = []
        for g in range(_CH // _D):
            ic = jnp.transpose(idx16[g : g + 1, :])
            slabs.append((ic == lanes).astype(jnp.bfloat16))
        oh = jnp.concatenate(slabs, axis=0)
        oh2 = jnp.concatenate([oh, oh], axis=1)
        so = jnp.dot(oh2, hilo, preferred_element_type=jnp.float32)
        rows = pl.ds(c * _CH, _CH)
        out_ref[rows, :] = z_ref[rows, :] * so[:, :_D] + so[:, _D:]


def kernel(z, y_gene, edges, running_mean, running_var, smoothed_mean, smoothed_var):
    n, d = z.shape

    sc_params = pltpu.CompilerParams()
    if "needs_layout_passes" in pltpu.CompilerParams.__dataclass_fields__:
        sc_params = dataclasses.replace(sc_params, needs_layout_passes=False)

    edges_padded = jnp.pad(edges, (0, 128 - (_NBINS + 1)), constant_values=jnp.inf)
    idx = pl.kernel(
        _bucketize_body,
        out_type=jax.ShapeDtypeStruct((n,), jnp.int32),
        mesh=plsc.VectorSubcoreMesh(core_axis_name="c", subcore_axis_name="s"),
        scratch_types=[
            pltpu.VMEM((n // _SC_WORKERS,), jnp.float32),
            pltpu.VMEM((128,), jnp.float32),
            pltpu.VMEM((n // _SC_WORKERS,), jnp.int32),
        ],
        compiler_params=sc_params,
    )(y_gene, edges_padded)

    pad = ((0, _BP - _NBINS), (0, 0))
    rm = jnp.pad(running_mean, pad)
    rv = jnp.pad(running_var, pad, constant_values=1.0)
    sm = jnp.pad(smoothed_mean, pad)
    sv = jnp.pad(smoothed_var, pad, constant_values=1.0)

    idx2d = idx.reshape(n // _D, _D)
    tbl_spec = pl.BlockSpec((_BP, _D), lambda i: (0, 0))
    out = pl.pallas_call(
        _main_body,
        grid=(n // _R,),
        in_specs=[
            pl.BlockSpec((_R // _D, _D), lambda i: (i, 0)),
            tbl_spec,
            tbl_spec,
            tbl_spec,
            tbl_spec,
            pl.BlockSpec((_R, _D), lambda i: (i, 0)),
        ],
        out_specs=pl.BlockSpec((_R, _D), lambda i: (i, 0)),
        out_shape=jax.ShapeDtypeStruct((n, d), jnp.float32),
        scratch_shapes=[pltpu.VMEM((2 * _BP, 2 * _D), jnp.bfloat16)],
    )(idx2d, rm, rv, sm, sv, z)
    return out
